# Initial kernel scaffold; baseline (speedup 1.0000x reference)
import jax
import jax.numpy as jnp
from jax.experimental import pallas as pl


def kernel(confidence, locations, labels, gt_locations):
    raise NotImplementedError("write your pallas kernel here")


# speedup vs baseline: 8.7357x; 8.7357x over previous
import functools

import jax
import jax.numpy as jnp
from jax import lax
from jax.experimental import pallas as pl
from jax.experimental.pallas import tpu as pltpu


def _mbl_body(c0_ref, c1_ref, c2_ref, lab_ref, diff_ref,
              mask_ref, acc_ref, nsel_ref, *, n_real, r):
    i = pl.program_id(0)

    c0 = c0_ref[...]
    c1 = c1_ref[...]
    c2 = c2_ref[...]
    m = jnp.maximum(jnp.maximum(c0, c1), c2)
    e0 = jnp.exp(c0 - m)
    e1 = jnp.exp(c1 - m)
    e2 = jnp.exp(c2 - m)
    logs = jnp.log(e0 + e1 + e2)
    bg = logs - (c0 - m)

    lab = lab_ref[...]
    col = lax.broadcasted_iota(jnp.int32, lab.shape, 1)
    valid = col < n_real
    pos = valid & (lab > 0)
    isneg = valid & (lab == 0)
    npos = jnp.sum(jnp.where(pos, 1, 0), axis=1, keepdims=True)
    k = npos * 3
    negcnt = jnp.sum(jnp.where(isneg, 1, 0), axis=1, keepdims=True)
    need = k < negcnt

    nsel_ref[...] = jnp.where(isneg, 1.0, 0.0)

    @pl.when(jnp.any(need))
    def _slow_path():
        bits = lax.bitcast_convert_type(bg, jnp.uint32)

        def pbody(t, p):
            b = 31 - t
            trial = p | (jnp.uint32(1) << jnp.uint32(b))
            cnt = jnp.sum(jnp.where(isneg & (bits >= trial), 1, 0),
                          axis=1, keepdims=True)
            return jnp.where(cnt >= k, trial, p)

        p = lax.fori_loop(0, 32, pbody, jnp.zeros((r, 1), jnp.uint32))
        gt = isneg & (bits > p)
        g = jnp.sum(jnp.where(gt, 1, 0), axis=1, keepdims=True)
        eq = isneg & (bits == p)
        eneed = k - g

        def tbody(t, tt):
            b = 13 - t
            trial = tt | (1 << b)
            c = jnp.sum(jnp.where(eq & (col < trial), 1, 0),
                        axis=1, keepdims=True)
            return jnp.where(c < eneed, trial, tt)

        tt = lax.fori_loop(0, 14, tbody, jnp.zeros((r, 1), jnp.int32))
        sel = gt | (eq & (col <= tt))
        combined = (need & sel) | (~need & isneg)
        nsel_ref[...] = jnp.where(combined, 1.0, 0.0)

    selneg = nsel_ref[...] > 0.5
    mask = pos | selneg
    mask_ref[...] = jnp.where(mask, 1, 0)
    maskf = jnp.where(mask, 1.0, 0.0)

    ce = jnp.where(lab == 0, bg,
                   jnp.where(lab == 1, logs - (c1 - m), logs - (c2 - m)))
    w = jnp.where(lab == 1, 2.0, 1.0)
    cls_sum = jnp.sum(ce * w * maskf)

    posf = jnp.where(pos, 1.0, 0.0)

    def sl1(d):
        ad = jnp.abs(d)
        return jnp.where(ad < 1.0, 0.5 * d * d, ad - 0.5)

    sl1s = (sl1(diff_ref[0]) + sl1(diff_ref[1])
            + sl1(diff_ref[2]) + sl1(diff_ref[3]))
    sl1_sum = jnp.sum(sl1s * posf)
    nposf = jnp.sum(posf)
    mws = jnp.sum(w * posf)

    @pl.when(i == 0)
    def _init():
        acc_ref[0] = 0.0
        acc_ref[1] = 0.0
        acc_ref[2] = 0.0
        acc_ref[3] = 0.0

    acc_ref[0] += sl1_sum
    acc_ref[1] += cls_sum
    acc_ref[2] += nposf
    acc_ref[3] += mws


def kernel(confidence, locations, labels, gt_locations):
    B, N, _ = confidence.shape
    R = 8
    NPAD = ((N + 127) // 128) * 128
    pad = NPAD - N

    conf_t = jnp.pad(jnp.moveaxis(confidence, 2, 0), ((0, 0), (0, 0), (0, pad)))
    c0, c1, c2 = conf_t[0], conf_t[1], conf_t[2]
    diff = jnp.pad(jnp.moveaxis(locations - gt_locations, 2, 0),
                   ((0, 0), (0, 0), (0, pad)))
    labp = jnp.pad(labels, ((0, 0), (0, pad)))

    mask_pad, acc = pl.pallas_call(
        functools.partial(_mbl_body, n_real=N, r=R),
        grid=(B // R,),
        in_specs=[
            pl.BlockSpec((R, NPAD), lambda i: (i, 0)),
            pl.BlockSpec((R, NPAD), lambda i: (i, 0)),
            pl.BlockSpec((R, NPAD), lambda i: (i, 0)),
            pl.BlockSpec((R, NPAD), lambda i: (i, 0)),
            pl.BlockSpec((4, R, NPAD), lambda i: (0, i, 0)),
        ],
        out_specs=[
            pl.BlockSpec((R, NPAD), lambda i: (i, 0)),
            pl.BlockSpec(memory_space=pltpu.SMEM),
        ],
        out_shape=[
            jax.ShapeDtypeStruct((B, NPAD), jnp.int32),
            jax.ShapeDtypeStruct((4,), jnp.float32),
        ],
        scratch_shapes=[pltpu.VMEM((R, NPAD), jnp.float32)],
    )(c0, c1, c2, labp, diff)

    mask = mask_pad[:, :N].astype(bool)
    return (acc[0] / acc[2], acc[1] / acc[3], mask)
